# Initial kernel scaffold; baseline (speedup 1.0000x reference)
import jax
import jax.numpy as jnp
from jax.experimental import pallas as pl


def kernel(gene_expression, gene_embeddings, pathway_embeddings, gene_pathway_mask, Wg, bg, Wp, bp, A1, a1b, A2, a2b, Wo, bo):
    raise NotImplementedError("write your pallas kernel here")


# speedup vs baseline: 2.1579x; 2.1579x over previous
import jax
import jax.numpy as jnp
from jax.experimental import pallas as pl
from jax.experimental.pallas import tpu as pltpu


def _kg_kernel(expr_ref, gembT_ref, pemb_ref, mask_ref,
               Wg_ref, bg_ref, WpT_ref, bp_ref,
               A1g_ref, A1pT_ref, a1b_ref, a2_ref, Wo_ref, bo_ref,
               out_ref,
               UT_ref, gsc_ref, c_ref):
    p = pl.program_id(0)

    @pl.when(p == 0)
    def _prep():
        g_projT = jnp.dot(Wg_ref[...], gembT_ref[...],
                          preferred_element_type=jnp.float32) + bg_ref[...]
        baseT = jnp.dot(A1g_ref[...], g_projT,
                        preferred_element_type=jnp.float32)
        expr = expr_ref[...]
        UT_ref[...] = expr[:, None, :] * baseT[None, :, :]
        w0 = jnp.dot(Wo_ref[...], g_projT,
                     preferred_element_type=jnp.float32)
        gsc_ref[...] = expr * w0
        p_proj = jnp.dot(pemb_ref[...], WpT_ref[...],
                         preferred_element_type=jnp.float32) + bp_ref[...]
        c_ref[...] = jnp.dot(p_proj, A1pT_ref[...],
                             preferred_element_type=jnp.float32) + a1b_ref[...]

    cp = c_ref[p, :]
    t = jnp.tanh(UT_ref[...] + cp[None, :, None])
    a2v = a2_ref[0, :]
    L = jnp.sum(t * a2v[None, :, None], axis=1)
    m = mask_ref[p, :]
    valid = (m > 0.0)[None, :]
    Lm = jnp.where(valid, L, jnp.float32(-1e30))
    rowmax = jnp.max(Lm, axis=1, keepdims=True)
    e = jnp.where(valid, jnp.exp(L - rowmax), 0.0)
    denom = jnp.sum(e, axis=1)
    num = jnp.sum(e * gsc_ref[...], axis=1)
    score = jnp.where(denom > 0.0, num / denom + bo_ref[0, 0], 0.0)
    out_ref[0, 0, :] = score


def kernel(gene_expression, gene_embeddings, pathway_embeddings,
           gene_pathway_mask, Wg, bg, Wp, bp, A1, a1b, A2, a2b, Wo, bo):
    B, G = gene_expression.shape
    P = pathway_embeddings.shape[0]
    H = Wg.shape[0]

    gembT = gene_embeddings.T
    A1g = A1[:, :H]
    A1pT = A1[:, H:].T
    WpT = Wp.T
    bg2 = bg.reshape(H, 1)
    bp2 = bp.reshape(1, H)
    a1b2 = a1b.reshape(1, H)
    bo2 = bo.reshape(1, 1)

    def full(x):
        return pl.BlockSpec(x.shape, lambda p, _nd=x.ndim: (0,) * _nd)

    ins = (gene_expression, gembT, pathway_embeddings, gene_pathway_mask,
           Wg, bg2, WpT, bp2, A1g, A1pT, a1b2, A2, Wo, bo2)

    out = pl.pallas_call(
        _kg_kernel,
        grid=(P,),
        in_specs=[full(x) for x in ins],
        out_specs=pl.BlockSpec((1, 1, B), lambda p: (p, 0, 0)),
        out_shape=jax.ShapeDtypeStruct((P, 1, B), jnp.float32),
        scratch_shapes=[
            pltpu.VMEM((B, H, G), jnp.float32),
            pltpu.VMEM((B, G), jnp.float32),
            pltpu.VMEM((P, H), jnp.float32),
        ],
    )(*ins)
    return out.reshape(P, B).T


# speedup vs baseline: 3.2725x; 1.5165x over previous
import jax
import jax.numpy as jnp
from jax.experimental import pallas as pl
from jax.experimental.pallas import tpu as pltpu


def _kg_kernel(expr_ref, gembT_ref, pemb_ref, mask_ref,
               Wg_ref, bg_ref, WpT_ref, bp_ref,
               A1g_ref, A1pT_ref, a1b_ref, a2_ref, Wo_ref, bo_ref,
               out_ref,
               UT_ref, gsc_ref, c_ref):
    p = pl.program_id(0)

    @pl.when(p == 0)
    def _prep():
        g_projT = jnp.dot(Wg_ref[...], gembT_ref[...],
                          preferred_element_type=jnp.float32) + bg_ref[...]
        baseT = jnp.dot(A1g_ref[...], g_projT,
                        preferred_element_type=jnp.float32)
        expr = expr_ref[...]
        UT_ref[...] = expr[:, None, :] * baseT[None, :, :]
        w0 = jnp.dot(Wo_ref[...], g_projT,
                     preferred_element_type=jnp.float32)
        gsc_ref[...] = expr * w0
        p_proj = jnp.dot(pemb_ref[...], WpT_ref[...],
                         preferred_element_type=jnp.float32) + bp_ref[...]
        c_ref[...] = jnp.dot(p_proj, A1pT_ref[...],
                             preferred_element_type=jnp.float32) + a1b_ref[...]

    cp = c_ref[p, :]
    t = jnp.tanh(UT_ref[...] + cp[None, :, None])
    a2r = a2_ref[...]
    L = jnp.concatenate(
        [jnp.dot(a2r, t[b], preferred_element_type=jnp.float32)
         for b in range(t.shape[0])], axis=0)
    m = mask_ref[p, :]
    valid = (m > 0.0)[None, :]
    Lm = jnp.where(valid, L, jnp.float32(-1e30))
    rowmax = jnp.max(Lm, axis=1, keepdims=True)
    e = jnp.where(valid, jnp.exp(L - rowmax), 0.0)
    denom = jnp.sum(e, axis=1)
    num = jnp.sum(e * gsc_ref[...], axis=1)
    score = jnp.where(denom > 0.0, num / denom + bo_ref[0, 0], 0.0)
    out_ref[0, 0, :] = score


def kernel(gene_expression, gene_embeddings, pathway_embeddings,
           gene_pathway_mask, Wg, bg, Wp, bp, A1, a1b, A2, a2b, Wo, bo):
    B, G = gene_expression.shape
    P = pathway_embeddings.shape[0]
    H = Wg.shape[0]

    gembT = gene_embeddings.T
    A1g = A1[:, :H]
    A1pT = A1[:, H:].T
    WpT = Wp.T
    bg2 = bg.reshape(H, 1)
    bp2 = bp.reshape(1, H)
    a1b2 = a1b.reshape(1, H)
    bo2 = bo.reshape(1, 1)

    def full(x):
        return pl.BlockSpec(x.shape, lambda p, _nd=x.ndim: (0,) * _nd)

    ins = (gene_expression, gembT, pathway_embeddings, gene_pathway_mask,
           Wg, bg2, WpT, bp2, A1g, A1pT, a1b2, A2, Wo, bo2)

    out = pl.pallas_call(
        _kg_kernel,
        grid=(P,),
        in_specs=[full(x) for x in ins],
        out_specs=pl.BlockSpec((1, 1, B), lambda p: (p, 0, 0)),
        out_shape=jax.ShapeDtypeStruct((P, 1, B), jnp.float32),
        scratch_shapes=[
            pltpu.VMEM((B, H, G), jnp.float32),
            pltpu.VMEM((B, G), jnp.float32),
            pltpu.VMEM((P, H), jnp.float32),
        ],
    )(*ins)
    return out.reshape(P, B).T
